# Initial kernel scaffold; baseline (speedup 1.0000x reference)
import jax
import jax.numpy as jnp
from jax.experimental import pallas as pl


def kernel(pos, edge_index, batch, lin0_W, lin0_b, conv_W, lin1_W, lin1_b, mlp1_W, mlp1_b, bn1_g, bn1_b, mlp2_W, mlp2_b, bn2_g, bn2_b, mlp3_W, mlp3_b):
    raise NotImplementedError("write your pallas kernel here")


# speedup vs baseline: 2.6358x; 2.6358x over previous
import functools

import numpy as np
import jax
import jax.numpy as jnp
from jax import lax
from jax.experimental import pallas as pl
from jax.experimental.pallas import tpu as pltpu
from jax.experimental.pallas import tpu_sc as plsc

N = 50000
E = 800000
H = 64
HH = 32
NLAYERS = 4
ALPHA = 0.1
THETA = 0.5
G = 32

R = 2000
NB = N // R

NSC = 2
NSUB = 16
CH = 128
CT = 391
EPT = CT * CH
E_PAD = EPT * NSUB
ACC_R = 51200
ZR = ACC_R // NSUB
OPT = 3128

def _dot(a, b):
    return jnp.dot(a, b, preferred_element_type=jnp.float32)


def _lin0_body(pos_ref, w_ref, b_ref, o_ref):
    c = pl.program_id(1)
    x = _dot(pos_ref[...], w_ref[...])
    x = jnp.maximum(x + b_ref[...], 0.0)
    o_ref[...] = jnp.where(c == 0, x[:, :HH], x[:, HH:])


def _lin0(pos, w, b):
    return pl.pallas_call(
        _lin0_body,
        grid=(NB, NSC),
        in_specs=[
            pl.BlockSpec((R, 3), lambda rb, c: (rb, 0)),
            pl.BlockSpec((3, H), lambda rb, c: (0, 0)),
            pl.BlockSpec((1, H), lambda rb, c: (0, 0)),
        ],
        out_specs=pl.BlockSpec((R, HH), lambda rb, c: (c * NB + rb, 0)),
        out_shape=jax.ShapeDtypeStruct((2 * N, HH), jnp.float32),
    )(pos, w, b.reshape(1, H))


def _segsum_body(x2, srcI, dstI, zs, out, src_v, dst_v, rows_v, acc, sem):
    c = lax.axis_index("c")
    s = lax.axis_index("s")
    pltpu.sync_copy(zs, acc.at[pl.ds(s * ZR, ZR)])
    plsc.subcore_barrier()

    def body(j, carry):
        pltpu.sync_copy(srcI.at[c, s, j], src_v)
        pltpu.sync_copy(dstI.at[s, j], dst_v)
        pltpu.async_copy(x2.at[src_v], rows_v, sem).wait()
        pltpu.sync_copy(rows_v, acc.at[dst_v], add=True)
        return carry

    lax.fori_loop(0, CT, body, 0)
    plsc.subcore_barrier()
    base = jnp.minimum(s * OPT, N - OPT)
    pltpu.sync_copy(acc.at[pl.ds(base, OPT)],
                    out.at[pl.ds(c * N + base, OPT)])


def _segsum(x2, srcI, dstI, zs):
    mesh = plsc.VectorSubcoreMesh(core_axis_name="c", subcore_axis_name="s")
    return pl.kernel(
        _segsum_body,
        out_type=jax.ShapeDtypeStruct((2 * N, HH), jnp.float32),
        mesh=mesh,
        scratch_types=[
            pltpu.VMEM((CH,), jnp.int32),
            pltpu.VMEM((CH,), jnp.int32),
            pltpu.VMEM((CH, HH), jnp.float32),
            pltpu.VMEM_SHARED((ACC_R, HH), jnp.float32),
            pltpu.SemaphoreType.DMA,
        ],
        compiler_params=pltpu.CompilerParams(use_tc_tiling_on_sc=False),
    )(x2, srcI, dstI, zs)


def _dense_body(beta, alo, ahi, xlo, xhi, w_ref, o_ref):
    c = pl.program_id(1)
    s_lo = (1.0 - ALPHA) * alo[...] + ALPHA * xlo[...]
    s_hi = (1.0 - ALPHA) * ahi[...] + ALPHA * xhi[...]
    s = jnp.concatenate([s_lo, s_hi], axis=1)
    w = w_ref[...]
    wh = jnp.where(c == 0, w[:, :HH], w[:, HH:])
    sh = jnp.where(c == 0, s_lo, s_hi)
    t = (1.0 - beta) * sh + beta * _dot(s, wh)
    o_ref[...] = jnp.maximum(t, 0.0)


def _dense(agg2, x02, w, beta):
    body = functools.partial(_dense_body, beta)
    return pl.pallas_call(
        body,
        grid=(NB, NSC),
        in_specs=[
            pl.BlockSpec((R, HH), lambda rb, c: (rb, 0)),
            pl.BlockSpec((R, HH), lambda rb, c: (NB + rb, 0)),
            pl.BlockSpec((R, HH), lambda rb, c: (rb, 0)),
            pl.BlockSpec((R, HH), lambda rb, c: (NB + rb, 0)),
            pl.BlockSpec((H, H), lambda rb, c: (0, 0)),
        ],
        out_specs=pl.BlockSpec((R, HH), lambda rb, c: (c * NB + rb, 0)),
        out_shape=jax.ShapeDtypeStruct((2 * N, HH), jnp.float32),
    )(agg2, agg2, x02, x02, w)


def _final_body(xlo, xhi, w_ref, b_ref, batch_ref, m1w, m1b, g1, c1,
                m2w, m2b, g2, c2, m3w, m3b, o_ref, pooled):
    rb = pl.program_id(0)
    x = jnp.concatenate([xlo[...], xhi[...]], axis=1)
    y = _dot(x, w_ref[...]) + b_ref[...]
    bid = batch_ref[0]
    gi = lax.broadcasted_iota(jnp.int32, (1, G), 1)
    mask = bid == gi
    neg = jnp.float32(-jnp.inf)
    cols = []
    for g in range(G):
        mg = mask[:, g:g + 1]
        cols.append(jnp.max(jnp.where(mg, y, neg), axis=0, keepdims=True))
    pm = jnp.concatenate(cols, axis=0)
    prev = jnp.where(rb == 0, jnp.full((G, H), neg, jnp.float32), pooled[...])
    pooled[...] = jnp.maximum(prev, pm)

    @pl.when(rb == 0)
    def _():
        o_ref[...] = jnp.zeros_like(o_ref)

    @pl.when(rb == NB - 1)
    def _():
        p = pooled[...]

        def bn(v, gg, bb):
            m = jnp.mean(v, axis=0, keepdims=True)
            var = jnp.mean((v - m) ** 2, axis=0, keepdims=True)
            return (v - m) / jnp.sqrt(var + 1e-5) * gg[...] + bb[...]

        h1 = jnp.maximum(bn(_dot(p, m1w[...])
                            + m1b[...], g1, c1), 0.0)
        h2 = jnp.maximum(bn(_dot(h1, m2w[...])
                            + m2b[...], g2, c2), 0.0)
        o = _dot(h2, m3w[...]) + m3b[...]
        mx = jnp.max(o, axis=1, keepdims=True)
        lse = jnp.log(jnp.sum(jnp.exp(o - mx), axis=1, keepdims=True)) + mx
        o_ref[...] = o - lse


def _final(x2, batch3, lin1_W, lin1_b, m1w, m1b, g1, c1, m2w, m2b, g2, c2,
           m3w, m3b):
    full = lambda a, b: pl.BlockSpec((a, b), lambda rb: (0, 0))
    return pl.pallas_call(
        _final_body,
        grid=(NB,),
        in_specs=[
            pl.BlockSpec((R, HH), lambda rb: (rb, 0)),
            pl.BlockSpec((R, HH), lambda rb: (NB + rb, 0)),
            full(H, H),
            full(1, H),
            pl.BlockSpec((1, R, 1), lambda rb: (rb, 0, 0)),
            full(H, H), full(1, H), full(1, H), full(1, H),
            full(H, H), full(1, H), full(1, H), full(1, H),
            full(H, 10), full(1, 10),
        ],
        out_specs=pl.BlockSpec((G, 10), lambda rb: (0, 0)),
        out_shape=jax.ShapeDtypeStruct((G, 10), jnp.float32),
        scratch_shapes=[pltpu.VMEM((G, H), jnp.float32)],
    )(x2, x2, lin1_W, lin1_b.reshape(1, H), batch3,
      m1w, m1b.reshape(1, H), g1.reshape(1, H), c1.reshape(1, H),
      m2w, m2b.reshape(1, H), g2.reshape(1, H), c2.reshape(1, H),
      m3w, m3b.reshape(1, 10))


def kernel(pos, edge_index, batch, lin0_W, lin0_b, conv_W, lin1_W, lin1_b,
           mlp1_W, mlp1_b, bn1_g, bn1_b, mlp2_W, mlp2_b, bn2_g, bn2_b,
           mlp3_W, mlp3_b):
    src = edge_index[0]
    dst = edge_index[1]
    pad = E_PAD - E
    srcp = jnp.concatenate([src, jnp.zeros((pad,), jnp.int32)])
    dstp = jnp.concatenate([dst, jnp.full((pad,), N, jnp.int32)])
    perm = jnp.argsort(dstp, stable=True)
    srcp = srcp[perm]
    dstp = dstp[perm]
    srcI = jnp.stack([srcp, srcp + N]).reshape(NSC, NSUB, CT, CH)
    dstI = dstp.reshape(NSUB, CT, CH)
    zs = jnp.zeros((ZR, HH), jnp.float32)
    batch3 = batch.reshape(NB, R, 1)

    x02 = _lin0(pos, lin0_W, lin0_b)
    x2 = x02
    for layer in range(NLAYERS):
        beta = float(np.log(THETA / (layer + 1) + 1.0))
        agg2 = _segsum(x2, srcI, dstI, zs)
        x2 = _dense(agg2, x02, conv_W[layer], beta)
    return _final(x2, batch3, lin1_W, lin1_b, mlp1_W, mlp1_b, bn1_g, bn1_b,
                  mlp2_W, mlp2_b, bn2_g, bn2_b, mlp3_W, mlp3_b)


# speedup vs baseline: 3.7018x; 1.4044x over previous
import functools

import numpy as np
import jax
import jax.numpy as jnp
from jax import lax
from jax.experimental import pallas as pl
from jax.experimental.pallas import tpu as pltpu
from jax.experimental.pallas import tpu_sc as plsc

N = 50000
E = 800000
H = 64
HH = 32
NLAYERS = 4
ALPHA = 0.1
THETA = 0.5
G = 32

R = 2000
NB = N // R

NSC = 2
NSUB = 16
CH = 128
SB = 8
CTO = 49
CT = SB * CTO
EPT = CT * CH
E_PAD = EPT * NSUB
ACC_R = 51200
ZR = ACC_R // NSUB
OPT = 3128

def _dot(a, b):
    return jnp.dot(a, b, preferred_element_type=jnp.float32)


def _lin0_body(pos_ref, w_ref, b_ref, o_ref):
    c = pl.program_id(1)
    x = _dot(pos_ref[...], w_ref[...])
    x = jnp.maximum(x + b_ref[...], 0.0)
    o_ref[...] = jnp.where(c == 0, x[:, :HH], x[:, HH:])


def _lin0(pos, w, b):
    return pl.pallas_call(
        _lin0_body,
        grid=(NB, NSC),
        in_specs=[
            pl.BlockSpec((R, 3), lambda rb, c: (rb, 0)),
            pl.BlockSpec((3, H), lambda rb, c: (0, 0)),
            pl.BlockSpec((1, H), lambda rb, c: (0, 0)),
        ],
        out_specs=pl.BlockSpec((R, HH), lambda rb, c: (c * NB + rb, 0)),
        out_shape=jax.ShapeDtypeStruct((2 * N, HH), jnp.float32),
    )(pos, w, b.reshape(1, H))


def _segsum_body(x2, srcI, dstI, zs, out, sidx, didx, buf0, buf1, acc,
                 sg0, sg1, ss0, ss1):
    c = lax.axis_index("c")
    s = lax.axis_index("s")
    pltpu.sync_copy(zs, acc.at[pl.ds(s * ZR, ZR)])
    plsc.subcore_barrier()

    bufs = (buf0, buf1)
    gsems = (sg0, sg1)
    ssems = (ss0, ss1)

    def outer(g, carry):
        pltpu.sync_copy(srcI.at[c, s, pl.ds(g * SB, SB)], sidx)
        pltpu.sync_copy(dstI.at[s, pl.ds(g * SB, SB)], didx)
        gd = {0: pltpu.async_copy(x2.at[sidx.at[0]], bufs[0], gsems[0])}
        sd = {}
        for w in range(SB):
            b = w % 2
            gd[w].wait()
            sd[w] = pltpu.async_copy(bufs[b], acc.at[didx.at[w]],
                                     ssems[b], add=True)
            if w + 1 < SB:
                if w >= 1:
                    sd[w - 1].wait()
                gd[w + 1] = pltpu.async_copy(x2.at[sidx.at[w + 1]],
                                             bufs[(w + 1) % 2],
                                             gsems[(w + 1) % 2])
        sd[SB - 2].wait()
        sd[SB - 1].wait()
        return carry

    lax.fori_loop(0, CTO, outer, 0)
    plsc.subcore_barrier()
    base = jnp.minimum(s * OPT, N - OPT)
    pltpu.sync_copy(acc.at[pl.ds(base, OPT)],
                    out.at[pl.ds(c * N + base, OPT)])


def _segsum(x2, srcI, dstI, zs):
    mesh = plsc.VectorSubcoreMesh(core_axis_name="c", subcore_axis_name="s")
    return pl.kernel(
        _segsum_body,
        out_type=jax.ShapeDtypeStruct((2 * N, HH), jnp.float32),
        mesh=mesh,
        scratch_types=[
            pltpu.VMEM((SB, CH), jnp.int32),
            pltpu.VMEM((SB, CH), jnp.int32),
            pltpu.VMEM((CH, HH), jnp.float32),
            pltpu.VMEM((CH, HH), jnp.float32),
            pltpu.VMEM_SHARED((ACC_R, HH), jnp.float32),
            pltpu.SemaphoreType.DMA,
            pltpu.SemaphoreType.DMA,
            pltpu.SemaphoreType.DMA,
            pltpu.SemaphoreType.DMA,
        ],
        compiler_params=pltpu.CompilerParams(use_tc_tiling_on_sc=False),
    )(x2, srcI, dstI, zs)


def _dense_body(beta, alo, ahi, xlo, xhi, w_ref, o_ref):
    c = pl.program_id(1)
    s_lo = (1.0 - ALPHA) * alo[...] + ALPHA * xlo[...]
    s_hi = (1.0 - ALPHA) * ahi[...] + ALPHA * xhi[...]
    s = jnp.concatenate([s_lo, s_hi], axis=1)
    w = w_ref[...]
    wh = jnp.where(c == 0, w[:, :HH], w[:, HH:])
    sh = jnp.where(c == 0, s_lo, s_hi)
    t = (1.0 - beta) * sh + beta * _dot(s, wh)
    o_ref[...] = jnp.maximum(t, 0.0)


def _dense(agg2, x02, w, beta):
    body = functools.partial(_dense_body, beta)
    return pl.pallas_call(
        body,
        grid=(NB, NSC),
        in_specs=[
            pl.BlockSpec((R, HH), lambda rb, c: (rb, 0)),
            pl.BlockSpec((R, HH), lambda rb, c: (NB + rb, 0)),
            pl.BlockSpec((R, HH), lambda rb, c: (rb, 0)),
            pl.BlockSpec((R, HH), lambda rb, c: (NB + rb, 0)),
            pl.BlockSpec((H, H), lambda rb, c: (0, 0)),
        ],
        out_specs=pl.BlockSpec((R, HH), lambda rb, c: (c * NB + rb, 0)),
        out_shape=jax.ShapeDtypeStruct((2 * N, HH), jnp.float32),
    )(agg2, agg2, x02, x02, w)


def _final_body(xlo, xhi, w_ref, b_ref, batch_ref, m1w, m1b, g1, c1,
                m2w, m2b, g2, c2, m3w, m3b, o_ref, pooled):
    rb = pl.program_id(0)
    x = jnp.concatenate([xlo[...], xhi[...]], axis=1)
    y = _dot(x, w_ref[...]) + b_ref[...]
    bid = batch_ref[0]
    gi = lax.broadcasted_iota(jnp.int32, (1, G), 1)
    mask = bid == gi
    neg = jnp.float32(-jnp.inf)
    cols = []
    for g in range(G):
        mg = mask[:, g:g + 1]
        cols.append(jnp.max(jnp.where(mg, y, neg), axis=0, keepdims=True))
    pm = jnp.concatenate(cols, axis=0)
    prev = jnp.where(rb == 0, jnp.full((G, H), neg, jnp.float32), pooled[...])
    pooled[...] = jnp.maximum(prev, pm)

    @pl.when(rb == 0)
    def _():
        o_ref[...] = jnp.zeros_like(o_ref)

    @pl.when(rb == NB - 1)
    def _():
        p = pooled[...]

        def bn(v, gg, bb):
            m = jnp.mean(v, axis=0, keepdims=True)
            var = jnp.mean((v - m) ** 2, axis=0, keepdims=True)
            return (v - m) / jnp.sqrt(var + 1e-5) * gg[...] + bb[...]

        h1 = jnp.maximum(bn(_dot(p, m1w[...])
                            + m1b[...], g1, c1), 0.0)
        h2 = jnp.maximum(bn(_dot(h1, m2w[...])
                            + m2b[...], g2, c2), 0.0)
        o = _dot(h2, m3w[...]) + m3b[...]
        mx = jnp.max(o, axis=1, keepdims=True)
        lse = jnp.log(jnp.sum(jnp.exp(o - mx), axis=1, keepdims=True)) + mx
        o_ref[...] = o - lse


def _final(x2, batch3, lin1_W, lin1_b, m1w, m1b, g1, c1, m2w, m2b, g2, c2,
           m3w, m3b):
    full = lambda a, b: pl.BlockSpec((a, b), lambda rb: (0, 0))
    return pl.pallas_call(
        _final_body,
        grid=(NB,),
        in_specs=[
            pl.BlockSpec((R, HH), lambda rb: (rb, 0)),
            pl.BlockSpec((R, HH), lambda rb: (NB + rb, 0)),
            full(H, H),
            full(1, H),
            pl.BlockSpec((1, R, 1), lambda rb: (rb, 0, 0)),
            full(H, H), full(1, H), full(1, H), full(1, H),
            full(H, H), full(1, H), full(1, H), full(1, H),
            full(H, 10), full(1, 10),
        ],
        out_specs=pl.BlockSpec((G, 10), lambda rb: (0, 0)),
        out_shape=jax.ShapeDtypeStruct((G, 10), jnp.float32),
        scratch_shapes=[pltpu.VMEM((G, H), jnp.float32)],
    )(x2, x2, lin1_W, lin1_b.reshape(1, H), batch3,
      m1w, m1b.reshape(1, H), g1.reshape(1, H), c1.reshape(1, H),
      m2w, m2b.reshape(1, H), g2.reshape(1, H), c2.reshape(1, H),
      m3w, m3b.reshape(1, 10))


def kernel(pos, edge_index, batch, lin0_W, lin0_b, conv_W, lin1_W, lin1_b,
           mlp1_W, mlp1_b, bn1_g, bn1_b, mlp2_W, mlp2_b, bn2_g, bn2_b,
           mlp3_W, mlp3_b):
    src = edge_index[0]
    dst = edge_index[1]
    pad = E_PAD - E
    srcp = jnp.concatenate([src, jnp.zeros((pad,), jnp.int32)])
    dstp = jnp.concatenate([dst, jnp.full((pad,), N, jnp.int32)])
    dstp, srcp = lax.sort((dstp, srcp), num_keys=1, is_stable=True)
    srcI = jnp.stack([srcp, srcp + N]).reshape(NSC, NSUB, CT, CH)
    dstI = dstp.reshape(NSUB, CT, CH)
    zs = jnp.zeros((ZR, HH), jnp.float32)
    batch3 = batch.reshape(NB, R, 1)

    x02 = _lin0(pos, lin0_W, lin0_b)
    x2 = x02
    for layer in range(NLAYERS):
        beta = float(np.log(THETA / (layer + 1) + 1.0))
        agg2 = _segsum(x2, srcI, dstI, zs)
        x2 = _dense(agg2, x02, conv_W[layer], beta)
    return _final(x2, batch3, lin1_W, lin1_b, mlp1_W, mlp1_b, bn1_g, bn1_b,
                  mlp2_W, mlp2_b, bn2_g, bn2_b, mlp3_W, mlp3_b)


# speedup vs baseline: 4.4294x; 1.1966x over previous
import functools

import numpy as np
import jax
import jax.numpy as jnp
from jax import lax
from jax.experimental import pallas as pl
from jax.experimental.pallas import tpu as pltpu
from jax.experimental.pallas import tpu_sc as plsc

N = 50000
E = 800000
H = 64
HH = 32
NLAYERS = 4
ALPHA = 0.1
THETA = 0.5
G = 32

R = 2000
NB = N // R

NSC = 2
NSUB = 16
CH = 128
SB = 8
CTO = 49
CT = SB * CTO
EPT = CT * CH
E_PAD = EPT * NSUB
ACC_R = 51200
ZR = ACC_R // NSUB
OPT = 3128

def _dot(a, b):
    return jnp.dot(a, b, preferred_element_type=jnp.float32)


def _lin0_body(pos_ref, w_ref, b_ref, o_ref):
    c = pl.program_id(1)
    x = _dot(pos_ref[...], w_ref[...])
    x = jnp.maximum(x + b_ref[...], 0.0)
    o_ref[...] = jnp.where(c == 0, x[:, :HH], x[:, HH:])


def _lin0(pos, w, b):
    return pl.pallas_call(
        _lin0_body,
        grid=(NB, NSC),
        in_specs=[
            pl.BlockSpec((R, 3), lambda rb, c: (rb, 0)),
            pl.BlockSpec((3, H), lambda rb, c: (0, 0)),
            pl.BlockSpec((1, H), lambda rb, c: (0, 0)),
        ],
        out_specs=pl.BlockSpec((R, HH), lambda rb, c: (c * NB + rb, 0)),
        out_shape=jax.ShapeDtypeStruct((2 * N, HH), jnp.float32),
    )(pos, w, b.reshape(1, H))


def _segsum_body(x2, srcI, dstI, zs, out, sidx, didx,
                 buf0, buf1, buf2, buf3, acc,
                 sg0, sg1, sg2, sg3, ss0, ss1, ss2, ss3):
    c = lax.axis_index("c")
    s = lax.axis_index("s")
    pltpu.sync_copy(zs, acc.at[pl.ds(s * ZR, ZR)])
    plsc.subcore_barrier()

    bufs = (buf0, buf1, buf2, buf3)
    gsems = (sg0, sg1, sg2, sg3)
    ssems = (ss0, ss1, ss2, ss3)

    def outer(g, carry):
        pltpu.sync_copy(srcI.at[c, s, pl.ds(g * SB, SB)], sidx)
        pltpu.sync_copy(dstI.at[s, pl.ds(g * SB, SB)], didx)
        gd = {w: pltpu.async_copy(x2.at[sidx.at[w]], bufs[w], gsems[w])
              for w in range(3)}
        sd = {}
        for w in range(SB):
            b = w % 4
            gd[w].wait()
            sd[w] = pltpu.async_copy(bufs[b], acc.at[didx.at[w]],
                                     ssems[b], add=True)
            nxt = w + 3
            if nxt < SB:
                if nxt - 4 >= 0:
                    sd[nxt - 4].wait()
                gd[nxt] = pltpu.async_copy(x2.at[sidx.at[nxt]],
                                           bufs[nxt % 4], gsems[nxt % 4])
        for k in range(SB - 4, SB):
            sd[k].wait()
        return carry

    lax.fori_loop(0, CTO, outer, 0)
    plsc.subcore_barrier()
    base = jnp.minimum(s * OPT, N - OPT)
    pltpu.sync_copy(acc.at[pl.ds(base, OPT)],
                    out.at[pl.ds(c * N + base, OPT)])


def _segsum(x2, srcI, dstI, zs):
    mesh = plsc.VectorSubcoreMesh(core_axis_name="c", subcore_axis_name="s")
    return pl.kernel(
        _segsum_body,
        out_type=jax.ShapeDtypeStruct((2 * N, HH), jnp.float32),
        mesh=mesh,
        scratch_types=[
            pltpu.VMEM((SB, CH), jnp.int32),
            pltpu.VMEM((SB, CH), jnp.int32),
            pltpu.VMEM((CH, HH), jnp.float32),
            pltpu.VMEM((CH, HH), jnp.float32),
            pltpu.VMEM((CH, HH), jnp.float32),
            pltpu.VMEM((CH, HH), jnp.float32),
            pltpu.VMEM_SHARED((ACC_R, HH), jnp.float32),
        ] + [pltpu.SemaphoreType.DMA] * 8,
        compiler_params=pltpu.CompilerParams(use_tc_tiling_on_sc=False),
    )(x2, srcI, dstI, zs)


def _dense_body(beta, alo, ahi, xlo, xhi, w_ref, o_ref):
    c = pl.program_id(1)
    s_lo = (1.0 - ALPHA) * alo[...] + ALPHA * xlo[...]
    s_hi = (1.0 - ALPHA) * ahi[...] + ALPHA * xhi[...]
    s = jnp.concatenate([s_lo, s_hi], axis=1)
    w = w_ref[...]
    wh = jnp.where(c == 0, w[:, :HH], w[:, HH:])
    sh = jnp.where(c == 0, s_lo, s_hi)
    t = (1.0 - beta) * sh + beta * _dot(s, wh)
    o_ref[...] = jnp.maximum(t, 0.0)


def _dense(agg2, x02, w, beta):
    body = functools.partial(_dense_body, beta)
    return pl.pallas_call(
        body,
        grid=(NB, NSC),
        in_specs=[
            pl.BlockSpec((R, HH), lambda rb, c: (rb, 0)),
            pl.BlockSpec((R, HH), lambda rb, c: (NB + rb, 0)),
            pl.BlockSpec((R, HH), lambda rb, c: (rb, 0)),
            pl.BlockSpec((R, HH), lambda rb, c: (NB + rb, 0)),
            pl.BlockSpec((H, H), lambda rb, c: (0, 0)),
        ],
        out_specs=pl.BlockSpec((R, HH), lambda rb, c: (c * NB + rb, 0)),
        out_shape=jax.ShapeDtypeStruct((2 * N, HH), jnp.float32),
    )(agg2, agg2, x02, x02, w)


def _final_body(xlo, xhi, w_ref, b_ref, batch_ref, m1w, m1b, g1, c1,
                m2w, m2b, g2, c2, m3w, m3b, o_ref, pooled):
    rb = pl.program_id(0)
    x = jnp.concatenate([xlo[...], xhi[...]], axis=1)
    y = _dot(x, w_ref[...]) + b_ref[...]
    bid = batch_ref[0]
    gi = lax.broadcasted_iota(jnp.int32, (1, G), 1)
    mask = bid == gi
    neg = jnp.float32(-jnp.inf)
    cols = []
    for g in range(G):
        mg = mask[:, g:g + 1]
        cols.append(jnp.max(jnp.where(mg, y, neg), axis=0, keepdims=True))
    pm = jnp.concatenate(cols, axis=0)
    prev = jnp.where(rb == 0, jnp.full((G, H), neg, jnp.float32), pooled[...])
    pooled[...] = jnp.maximum(prev, pm)

    @pl.when(rb == 0)
    def _():
        o_ref[...] = jnp.zeros_like(o_ref)

    @pl.when(rb == NB - 1)
    def _():
        p = pooled[...]

        def bn(v, gg, bb):
            m = jnp.mean(v, axis=0, keepdims=True)
            var = jnp.mean((v - m) ** 2, axis=0, keepdims=True)
            return (v - m) / jnp.sqrt(var + 1e-5) * gg[...] + bb[...]

        h1 = jnp.maximum(bn(_dot(p, m1w[...])
                            + m1b[...], g1, c1), 0.0)
        h2 = jnp.maximum(bn(_dot(h1, m2w[...])
                            + m2b[...], g2, c2), 0.0)
        o = _dot(h2, m3w[...]) + m3b[...]
        mx = jnp.max(o, axis=1, keepdims=True)
        lse = jnp.log(jnp.sum(jnp.exp(o - mx), axis=1, keepdims=True)) + mx
        o_ref[...] = o - lse


def _final(x2, batch3, lin1_W, lin1_b, m1w, m1b, g1, c1, m2w, m2b, g2, c2,
           m3w, m3b):
    full = lambda a, b: pl.BlockSpec((a, b), lambda rb: (0, 0))
    return pl.pallas_call(
        _final_body,
        grid=(NB,),
        in_specs=[
            pl.BlockSpec((R, HH), lambda rb: (rb, 0)),
            pl.BlockSpec((R, HH), lambda rb: (NB + rb, 0)),
            full(H, H),
            full(1, H),
            pl.BlockSpec((1, R, 1), lambda rb: (rb, 0, 0)),
            full(H, H), full(1, H), full(1, H), full(1, H),
            full(H, H), full(1, H), full(1, H), full(1, H),
            full(H, 10), full(1, 10),
        ],
        out_specs=pl.BlockSpec((G, 10), lambda rb: (0, 0)),
        out_shape=jax.ShapeDtypeStruct((G, 10), jnp.float32),
        scratch_shapes=[pltpu.VMEM((G, H), jnp.float32)],
    )(x2, x2, lin1_W, lin1_b.reshape(1, H), batch3,
      m1w, m1b.reshape(1, H), g1.reshape(1, H), c1.reshape(1, H),
      m2w, m2b.reshape(1, H), g2.reshape(1, H), c2.reshape(1, H),
      m3w, m3b.reshape(1, 10))


def kernel(pos, edge_index, batch, lin0_W, lin0_b, conv_W, lin1_W, lin1_b,
           mlp1_W, mlp1_b, bn1_g, bn1_b, mlp2_W, mlp2_b, bn2_g, bn2_b,
           mlp3_W, mlp3_b):
    src = edge_index[0]
    dst = edge_index[1]
    pad = E_PAD - E
    srcp = jnp.concatenate([src, jnp.zeros((pad,), jnp.int32)])
    dstp = jnp.concatenate([dst, jnp.full((pad,), N, jnp.int32)])
    dstp, srcp = lax.sort((dstp, srcp), num_keys=1, is_stable=True)
    srcI = jnp.stack([srcp, srcp + N]).reshape(NSC, NSUB, CT, CH)
    dstI = dstp.reshape(NSUB, CT, CH)
    zs = jnp.zeros((ZR, HH), jnp.float32)
    batch3 = batch.reshape(NB, R, 1)

    x02 = _lin0(pos, lin0_W, lin0_b)
    x2 = x02
    for layer in range(NLAYERS):
        beta = float(np.log(THETA / (layer + 1) + 1.0))
        agg2 = _segsum(x2, srcI, dstI, zs)
        x2 = _dense(agg2, x02, conv_W[layer], beta)
    return _final(x2, batch3, lin1_W, lin1_b, mlp1_W, mlp1_b, bn1_g, bn1_b,
                  mlp2_W, mlp2_b, bn2_g, bn2_b, mlp3_W, mlp3_b)


# speedup vs baseline: 4.4631x; 1.0076x over previous
import functools

import numpy as np
import jax
import jax.numpy as jnp
from jax import lax
from jax.experimental import pallas as pl
from jax.experimental.pallas import tpu as pltpu
from jax.experimental.pallas import tpu_sc as plsc

N = 50000
E = 800000
H = 64
HH = 32
NLAYERS = 4
ALPHA = 0.1
THETA = 0.5
G = 32

R = 2000
NB = N // R

NSC = 2
NSUB = 16
CH = 128
SB = 8
CTO = 49
CT = SB * CTO
EPT = CT * CH
E_PAD = EPT * NSUB
ACC_R = 51200
ZR = ACC_R // NSUB
OPT = 3128

def _dot(a, b):
    return jnp.dot(a, b, preferred_element_type=jnp.float32)


def _lin0_body(pos_ref, w_ref, b_ref, o_ref):
    c = pl.program_id(1)
    x = _dot(pos_ref[...], w_ref[...])
    x = jnp.maximum(x + b_ref[...], 0.0)
    o_ref[...] = jnp.where(c == 0, x[:, :HH], x[:, HH:])


def _lin0(pos, w, b):
    return pl.pallas_call(
        _lin0_body,
        grid=(NB, NSC),
        in_specs=[
            pl.BlockSpec((R, 3), lambda rb, c: (rb, 0)),
            pl.BlockSpec((3, H), lambda rb, c: (0, 0)),
            pl.BlockSpec((1, H), lambda rb, c: (0, 0)),
        ],
        out_specs=pl.BlockSpec((R, HH), lambda rb, c: (c * NB + rb, 0)),
        out_shape=jax.ShapeDtypeStruct((2 * N, HH), jnp.float32),
    )(pos, w, b.reshape(1, H))


def _segsum_body(x2, srcI, dstI, zs, out, sidx, didx,
                 buf0, buf1, buf2, buf3, acc,
                 sg0, sg1, sg2, sg3, ss0, ss1, ss2, ss3):
    c = lax.axis_index("c")
    s = lax.axis_index("s")
    pltpu.sync_copy(zs, acc.at[pl.ds(s * ZR, ZR)])
    plsc.subcore_barrier()

    bufs = (buf0, buf1, buf2, buf3)
    gsems = (sg0, sg1, sg2, sg3)
    ssems = (ss0, ss1, ss2, ss3)

    def outer(g, carry):
        pltpu.sync_copy(srcI.at[c, s, pl.ds(g * SB, SB)], sidx)
        pltpu.sync_copy(dstI.at[s, pl.ds(g * SB, SB)], didx)
        gd = {w: pltpu.async_copy(x2.at[sidx.at[w]], bufs[w], gsems[w])
              for w in range(3)}
        sd = {}
        for w in range(SB):
            b = w % 4
            gd[w].wait()
            if w >= 1:
                sd[w - 1].wait()
            sd[w] = pltpu.async_copy(bufs[b], acc.at[didx.at[w]],
                                     ssems[b], add=True)
            nxt = w + 3
            if nxt < SB:
                gd[nxt] = pltpu.async_copy(x2.at[sidx.at[nxt]],
                                           bufs[nxt % 4], gsems[nxt % 4])
        sd[SB - 1].wait()
        return carry

    lax.fori_loop(0, CTO, outer, 0)
    plsc.subcore_barrier()
    base = jnp.minimum(s * OPT, N - OPT)
    pltpu.sync_copy(acc.at[pl.ds(base, OPT)],
                    out.at[pl.ds(c * N + base, OPT)])


def _segsum(x2, srcI, dstI, zs):
    mesh = plsc.VectorSubcoreMesh(core_axis_name="c", subcore_axis_name="s")
    return pl.kernel(
        _segsum_body,
        out_type=jax.ShapeDtypeStruct((2 * N, HH), jnp.float32),
        mesh=mesh,
        scratch_types=[
            pltpu.VMEM((SB, CH), jnp.int32),
            pltpu.VMEM((SB, CH), jnp.int32),
            pltpu.VMEM((CH, HH), jnp.float32),
            pltpu.VMEM((CH, HH), jnp.float32),
            pltpu.VMEM((CH, HH), jnp.float32),
            pltpu.VMEM((CH, HH), jnp.float32),
            pltpu.VMEM_SHARED((ACC_R, HH), jnp.float32),
        ] + [pltpu.SemaphoreType.DMA] * 8,
        compiler_params=pltpu.CompilerParams(use_tc_tiling_on_sc=False),
    )(x2, srcI, dstI, zs)


def _dense_body(beta, alo, ahi, xlo, xhi, w_ref, o_ref):
    c = pl.program_id(1)
    s_lo = (1.0 - ALPHA) * alo[...] + ALPHA * xlo[...]
    s_hi = (1.0 - ALPHA) * ahi[...] + ALPHA * xhi[...]
    s = jnp.concatenate([s_lo, s_hi], axis=1)
    w = w_ref[...]
    wh = jnp.where(c == 0, w[:, :HH], w[:, HH:])
    sh = jnp.where(c == 0, s_lo, s_hi)
    t = (1.0 - beta) * sh + beta * _dot(s, wh)
    o_ref[...] = jnp.maximum(t, 0.0)


def _dense(agg2, x02, w, beta):
    body = functools.partial(_dense_body, beta)
    return pl.pallas_call(
        body,
        grid=(NB, NSC),
        in_specs=[
            pl.BlockSpec((R, HH), lambda rb, c: (rb, 0)),
            pl.BlockSpec((R, HH), lambda rb, c: (NB + rb, 0)),
            pl.BlockSpec((R, HH), lambda rb, c: (rb, 0)),
            pl.BlockSpec((R, HH), lambda rb, c: (NB + rb, 0)),
            pl.BlockSpec((H, H), lambda rb, c: (0, 0)),
        ],
        out_specs=pl.BlockSpec((R, HH), lambda rb, c: (c * NB + rb, 0)),
        out_shape=jax.ShapeDtypeStruct((2 * N, HH), jnp.float32),
    )(agg2, agg2, x02, x02, w)


def _final_body(xlo, xhi, w_ref, b_ref, batch_ref, m1w, m1b, g1, c1,
                m2w, m2b, g2, c2, m3w, m3b, o_ref, pooled):
    rb = pl.program_id(0)
    x = jnp.concatenate([xlo[...], xhi[...]], axis=1)
    y = _dot(x, w_ref[...]) + b_ref[...]
    bid = batch_ref[0]
    gi = lax.broadcasted_iota(jnp.int32, (1, G), 1)
    mask = bid == gi
    neg = jnp.float32(-jnp.inf)
    cols = []
    for g in range(G):
        mg = mask[:, g:g + 1]
        cols.append(jnp.max(jnp.where(mg, y, neg), axis=0, keepdims=True))
    pm = jnp.concatenate(cols, axis=0)
    prev = jnp.where(rb == 0, jnp.full((G, H), neg, jnp.float32), pooled[...])
    pooled[...] = jnp.maximum(prev, pm)

    @pl.when(rb == 0)
    def _():
        o_ref[...] = jnp.zeros_like(o_ref)

    @pl.when(rb == NB - 1)
    def _():
        p = pooled[...]

        def bn(v, gg, bb):
            m = jnp.mean(v, axis=0, keepdims=True)
            var = jnp.mean((v - m) ** 2, axis=0, keepdims=True)
            return (v - m) / jnp.sqrt(var + 1e-5) * gg[...] + bb[...]

        h1 = jnp.maximum(bn(_dot(p, m1w[...])
                            + m1b[...], g1, c1), 0.0)
        h2 = jnp.maximum(bn(_dot(h1, m2w[...])
                            + m2b[...], g2, c2), 0.0)
        o = _dot(h2, m3w[...]) + m3b[...]
        mx = jnp.max(o, axis=1, keepdims=True)
        lse = jnp.log(jnp.sum(jnp.exp(o - mx), axis=1, keepdims=True)) + mx
        o_ref[...] = o - lse


def _final(x2, batch3, lin1_W, lin1_b, m1w, m1b, g1, c1, m2w, m2b, g2, c2,
           m3w, m3b):
    full = lambda a, b: pl.BlockSpec((a, b), lambda rb: (0, 0))
    return pl.pallas_call(
        _final_body,
        grid=(NB,),
        in_specs=[
            pl.BlockSpec((R, HH), lambda rb: (rb, 0)),
            pl.BlockSpec((R, HH), lambda rb: (NB + rb, 0)),
            full(H, H),
            full(1, H),
            pl.BlockSpec((1, R, 1), lambda rb: (rb, 0, 0)),
            full(H, H), full(1, H), full(1, H), full(1, H),
            full(H, H), full(1, H), full(1, H), full(1, H),
            full(H, 10), full(1, 10),
        ],
        out_specs=pl.BlockSpec((G, 10), lambda rb: (0, 0)),
        out_shape=jax.ShapeDtypeStruct((G, 10), jnp.float32),
        scratch_shapes=[pltpu.VMEM((G, H), jnp.float32)],
    )(x2, x2, lin1_W, lin1_b.reshape(1, H), batch3,
      m1w, m1b.reshape(1, H), g1.reshape(1, H), c1.reshape(1, H),
      m2w, m2b.reshape(1, H), g2.reshape(1, H), c2.reshape(1, H),
      m3w, m3b.reshape(1, 10))


def kernel(pos, edge_index, batch, lin0_W, lin0_b, conv_W, lin1_W, lin1_b,
           mlp1_W, mlp1_b, bn1_g, bn1_b, mlp2_W, mlp2_b, bn2_g, bn2_b,
           mlp3_W, mlp3_b):
    src = edge_index[0]
    dst = edge_index[1]
    pad = E_PAD - E
    srcp = jnp.concatenate([src, jnp.zeros((pad,), jnp.int32)])
    dstp = jnp.concatenate([dst, jnp.full((pad,), N, jnp.int32)])
    dstp, srcp = lax.sort((dstp, srcp), num_keys=1, is_stable=True)
    srcI = jnp.stack([srcp, srcp + N]).reshape(NSC, NSUB, CT, CH)
    dstI = dstp.reshape(NSUB, CT, CH)
    zs = jnp.zeros((ZR, HH), jnp.float32)
    batch3 = batch.reshape(NB, R, 1)

    x02 = _lin0(pos, lin0_W, lin0_b)
    x2 = x02
    for layer in range(NLAYERS):
        beta = float(np.log(THETA / (layer + 1) + 1.0))
        agg2 = _segsum(x2, srcI, dstI, zs)
        x2 = _dense(agg2, x02, conv_W[layer], beta)
    return _final(x2, batch3, lin1_W, lin1_b, mlp1_W, mlp1_b, bn1_g, bn1_b,
                  mlp2_W, mlp2_b, bn2_g, bn2_b, mlp3_W, mlp3_b)


# speedup vs baseline: 4.5958x; 1.0297x over previous
import functools

import numpy as np
import jax
import jax.numpy as jnp
from jax import lax
from jax.experimental import pallas as pl
from jax.experimental.pallas import tpu as pltpu
from jax.experimental.pallas import tpu_sc as plsc

N = 50000
E = 800000
H = 64
HH = 32
NLAYERS = 4
ALPHA = 0.1
THETA = 0.5
G = 32

R = 5000
NB = N // R

NSC = 2
NSUB = 16
CH = 128
SB = 8
CTO = 49
CT = SB * CTO
EPT = CT * CH
E_PAD = EPT * NSUB
ACC_R = 51200
ZR = ACC_R // NSUB
OPT = 3128

def _dot(a, b):
    return jnp.dot(a, b, preferred_element_type=jnp.float32)


def _lin0_body(pos_ref, w_ref, b_ref, o_ref):
    c = pl.program_id(1)
    x = _dot(pos_ref[...], w_ref[...])
    x = jnp.maximum(x + b_ref[...], 0.0)
    o_ref[...] = jnp.where(c == 0, x[:, :HH], x[:, HH:])


def _lin0(pos, w, b):
    return pl.pallas_call(
        _lin0_body,
        grid=(NB, NSC),
        in_specs=[
            pl.BlockSpec((R, 3), lambda rb, c: (rb, 0)),
            pl.BlockSpec((3, H), lambda rb, c: (0, 0)),
            pl.BlockSpec((1, H), lambda rb, c: (0, 0)),
        ],
        out_specs=pl.BlockSpec((R, HH), lambda rb, c: (c * NB + rb, 0)),
        out_shape=jax.ShapeDtypeStruct((2 * N, HH), jnp.float32),
    )(pos, w, b.reshape(1, H))


def _segsum_body(x2, srcI, dstI, zs, out, sidx, didx,
                 buf0, buf1, buf2, buf3, acc,
                 sg0, sg1, sg2, sg3, ss0, ss1, ss2, ss3):
    c = lax.axis_index("c")
    s = lax.axis_index("s")
    pltpu.sync_copy(zs, acc.at[pl.ds(s * ZR, ZR)])
    plsc.subcore_barrier()

    bufs = (buf0, buf1, buf2, buf3)
    gsems = (sg0, sg1, sg2, sg3)
    ssems = (ss0, ss1, ss2, ss3)

    def outer(g, carry):
        pltpu.sync_copy(srcI.at[c, s, pl.ds(g * SB, SB)], sidx)
        pltpu.sync_copy(dstI.at[s, pl.ds(g * SB, SB)], didx)
        gd = {w: pltpu.async_copy(x2.at[sidx.at[w]], bufs[w], gsems[w])
              for w in range(3)}
        sd = {}
        for w in range(SB):
            b = w % 4
            gd[w].wait()
            if w >= 1:
                sd[w - 1].wait()
            sd[w] = pltpu.async_copy(bufs[b], acc.at[didx.at[w]],
                                     ssems[b], add=True)
            nxt = w + 3
            if nxt < SB:
                gd[nxt] = pltpu.async_copy(x2.at[sidx.at[nxt]],
                                           bufs[nxt % 4], gsems[nxt % 4])
        sd[SB - 1].wait()
        return carry

    lax.fori_loop(0, CTO, outer, 0)
    plsc.subcore_barrier()
    base = jnp.minimum(s * OPT, N - OPT)
    pltpu.sync_copy(acc.at[pl.ds(base, OPT)],
                    out.at[pl.ds(c * N + base, OPT)])


def _segsum(x2, srcI, dstI, zs):
    mesh = plsc.VectorSubcoreMesh(core_axis_name="c", subcore_axis_name="s")
    return pl.kernel(
        _segsum_body,
        out_type=jax.ShapeDtypeStruct((2 * N, HH), jnp.float32),
        mesh=mesh,
        scratch_types=[
            pltpu.VMEM((SB, CH), jnp.int32),
            pltpu.VMEM((SB, CH), jnp.int32),
            pltpu.VMEM((CH, HH), jnp.float32),
            pltpu.VMEM((CH, HH), jnp.float32),
            pltpu.VMEM((CH, HH), jnp.float32),
            pltpu.VMEM((CH, HH), jnp.float32),
            pltpu.VMEM_SHARED((ACC_R, HH), jnp.float32),
        ] + [pltpu.SemaphoreType.DMA] * 8,
        compiler_params=pltpu.CompilerParams(use_tc_tiling_on_sc=False),
    )(x2, srcI, dstI, zs)


def _dense_body(beta, alo, ahi, xlo, xhi, w_ref, o_ref):
    c = pl.program_id(1)
    s_lo = (1.0 - ALPHA) * alo[...] + ALPHA * xlo[...]
    s_hi = (1.0 - ALPHA) * ahi[...] + ALPHA * xhi[...]
    s = jnp.concatenate([s_lo, s_hi], axis=1)
    w = w_ref[...]
    wh = jnp.where(c == 0, w[:, :HH], w[:, HH:])
    sh = jnp.where(c == 0, s_lo, s_hi)
    t = (1.0 - beta) * sh + beta * _dot(s, wh)
    o_ref[...] = jnp.maximum(t, 0.0)


def _dense(agg2, x02, w, beta):
    body = functools.partial(_dense_body, beta)
    return pl.pallas_call(
        body,
        grid=(NB, NSC),
        in_specs=[
            pl.BlockSpec((R, HH), lambda rb, c: (rb, 0)),
            pl.BlockSpec((R, HH), lambda rb, c: (NB + rb, 0)),
            pl.BlockSpec((R, HH), lambda rb, c: (rb, 0)),
            pl.BlockSpec((R, HH), lambda rb, c: (NB + rb, 0)),
            pl.BlockSpec((H, H), lambda rb, c: (0, 0)),
        ],
        out_specs=pl.BlockSpec((R, HH), lambda rb, c: (c * NB + rb, 0)),
        out_shape=jax.ShapeDtypeStruct((2 * N, HH), jnp.float32),
    )(agg2, agg2, x02, x02, w)


def _final_body(xlo, xhi, w_ref, b_ref, batch_ref, m1w, m1b, g1, c1,
                m2w, m2b, g2, c2, m3w, m3b, o_ref, pooled):
    rb = pl.program_id(0)
    x = jnp.concatenate([xlo[...], xhi[...]], axis=1)
    y = _dot(x, w_ref[...]) + b_ref[...]
    bid = batch_ref[0]
    gi = lax.broadcasted_iota(jnp.int32, (1, G), 1)
    mask = bid == gi
    neg = jnp.float32(-jnp.inf)
    cols = []
    for g in range(G):
        mg = mask[:, g:g + 1]
        cols.append(jnp.max(jnp.where(mg, y, neg), axis=0, keepdims=True))
    pm = jnp.concatenate(cols, axis=0)
    prev = jnp.where(rb == 0, jnp.full((G, H), neg, jnp.float32), pooled[...])
    pooled[...] = jnp.maximum(prev, pm)

    @pl.when(rb == 0)
    def _():
        o_ref[...] = jnp.zeros_like(o_ref)

    @pl.when(rb == NB - 1)
    def _():
        p = pooled[...]

        def bn(v, gg, bb):
            m = jnp.mean(v, axis=0, keepdims=True)
            var = jnp.mean((v - m) ** 2, axis=0, keepdims=True)
            return (v - m) / jnp.sqrt(var + 1e-5) * gg[...] + bb[...]

        h1 = jnp.maximum(bn(_dot(p, m1w[...])
                            + m1b[...], g1, c1), 0.0)
        h2 = jnp.maximum(bn(_dot(h1, m2w[...])
                            + m2b[...], g2, c2), 0.0)
        o = _dot(h2, m3w[...]) + m3b[...]
        mx = jnp.max(o, axis=1, keepdims=True)
        lse = jnp.log(jnp.sum(jnp.exp(o - mx), axis=1, keepdims=True)) + mx
        o_ref[...] = o - lse


def _final(x2, batch3, lin1_W, lin1_b, m1w, m1b, g1, c1, m2w, m2b, g2, c2,
           m3w, m3b):
    full = lambda a, b: pl.BlockSpec((a, b), lambda rb: (0, 0))
    return pl.pallas_call(
        _final_body,
        grid=(NB,),
        in_specs=[
            pl.BlockSpec((R, HH), lambda rb: (rb, 0)),
            pl.BlockSpec((R, HH), lambda rb: (NB + rb, 0)),
            full(H, H),
            full(1, H),
            pl.BlockSpec((1, R, 1), lambda rb: (rb, 0, 0)),
            full(H, H), full(1, H), full(1, H), full(1, H),
            full(H, H), full(1, H), full(1, H), full(1, H),
            full(H, 10), full(1, 10),
        ],
        out_specs=pl.BlockSpec((G, 10), lambda rb: (0, 0)),
        out_shape=jax.ShapeDtypeStruct((G, 10), jnp.float32),
        scratch_shapes=[pltpu.VMEM((G, H), jnp.float32)],
    )(x2, x2, lin1_W, lin1_b.reshape(1, H), batch3,
      m1w, m1b.reshape(1, H), g1.reshape(1, H), c1.reshape(1, H),
      m2w, m2b.reshape(1, H), g2.reshape(1, H), c2.reshape(1, H),
      m3w, m3b.reshape(1, 10))


def kernel(pos, edge_index, batch, lin0_W, lin0_b, conv_W, lin1_W, lin1_b,
           mlp1_W, mlp1_b, bn1_g, bn1_b, mlp2_W, mlp2_b, bn2_g, bn2_b,
           mlp3_W, mlp3_b):
    src = edge_index[0]
    dst = edge_index[1]
    pad = E_PAD - E
    srcp = jnp.concatenate([src, jnp.zeros((pad,), jnp.int32)])
    dstp = jnp.concatenate([dst, jnp.full((pad,), N, jnp.int32)])
    dstp, srcp = lax.sort((dstp, srcp), num_keys=1, is_stable=True)
    srcI = jnp.stack([srcp, srcp + N]).reshape(NSC, NSUB, CT, CH)
    dstI = dstp.reshape(NSUB, CT, CH)
    zs = jnp.zeros((ZR, HH), jnp.float32)
    batch3 = batch.reshape(NB, R, 1)

    x02 = _lin0(pos, lin0_W, lin0_b)
    x2 = x02
    for layer in range(NLAYERS):
        beta = float(np.log(THETA / (layer + 1) + 1.0))
        agg2 = _segsum(x2, srcI, dstI, zs)
        x2 = _dense(agg2, x02, conv_W[layer], beta)
    return _final(x2, batch3, lin1_W, lin1_b, mlp1_W, mlp1_b, bn1_g, bn1_b,
                  mlp2_W, mlp2_b, bn2_g, bn2_b, mlp3_W, mlp3_b)


# speedup vs baseline: 4.9065x; 1.0676x over previous
import functools

import numpy as np
import jax
import jax.numpy as jnp
from jax import lax
from jax.experimental import pallas as pl
from jax.experimental.pallas import tpu as pltpu
from jax.experimental.pallas import tpu_sc as plsc

N = 50000
E = 800000
H = 64
HH = 32
NLAYERS = 4
ALPHA = 0.1
THETA = 0.5
G = 32

R = 5000
NB = N // R

NSC = 2
NSUB = 16
CH = 128
SB = 8
CTO = 49
CT = SB * CTO
EPT = CT * CH
E_PAD = EPT * NSUB
ACC_R = 51200
ZR = ACC_R // NSUB
OPT = 3128

def _dot(a, b):
    return jnp.dot(a, b, preferred_element_type=jnp.float32)


def _lin0_body(pos_ref, w_ref, b_ref, o_ref):
    c = pl.program_id(1)
    x = _dot(pos_ref[...], w_ref[...])
    x = jnp.maximum(x + b_ref[...], 0.0)
    o_ref[...] = jnp.where(c == 0, x[:, :HH], x[:, HH:])


def _lin0(pos, w, b):
    return pl.pallas_call(
        _lin0_body,
        grid=(NB, NSC),
        in_specs=[
            pl.BlockSpec((R, 3), lambda rb, c: (rb, 0)),
            pl.BlockSpec((3, H), lambda rb, c: (0, 0)),
            pl.BlockSpec((1, H), lambda rb, c: (0, 0)),
        ],
        out_specs=pl.BlockSpec((R, HH), lambda rb, c: (c * NB + rb, 0)),
        out_shape=jax.ShapeDtypeStruct((2 * N, HH), jnp.float32),
    )(pos, w, b.reshape(1, H))


def _segsum_body(x2, srcI, dstI, zs, out, sidx, didx,
                 buf0, buf1, buf2, buf3, acc,
                 sg0, sg1, sg2, sg3, ss0, ss1, ss2, ss3):
    c = lax.axis_index("c")
    s = lax.axis_index("s")
    pltpu.sync_copy(zs, acc.at[pl.ds(s * ZR, ZR)])
    plsc.subcore_barrier()

    bufs = (buf0, buf1, buf2, buf3)
    gsems = (sg0, sg1, sg2, sg3)
    ssems = (ss0, ss1, ss2, ss3)

    def outer(g, carry):
        pltpu.sync_copy(srcI.at[c, s, pl.ds(g * SB, SB)], sidx)
        pltpu.sync_copy(dstI.at[s, pl.ds(g * SB, SB)], didx)
        gd = {w: pltpu.async_copy(x2.at[sidx.at[w]], bufs[w], gsems[w])
              for w in range(3)}
        sd = {}
        for w in range(SB):
            b = w % 4
            gd[w].wait()
            if w >= 1:
                sd[w - 1].wait()
            sd[w] = pltpu.async_copy(bufs[b], acc.at[didx.at[w]],
                                     ssems[b], add=True)
            nxt = w + 3
            if nxt < SB:
                gd[nxt] = pltpu.async_copy(x2.at[sidx.at[nxt]],
                                           bufs[nxt % 4], gsems[nxt % 4])
        sd[SB - 1].wait()
        return carry

    lax.fori_loop(0, CTO, outer, 0)
    plsc.subcore_barrier()
    base = jnp.minimum(s * OPT, N - OPT)
    pltpu.sync_copy(acc.at[pl.ds(base, OPT)],
                    out.at[pl.ds(c * N + base, OPT)])


def _segsum(x2, srcI, dstI, zs):
    mesh = plsc.VectorSubcoreMesh(core_axis_name="c", subcore_axis_name="s")
    return pl.kernel(
        _segsum_body,
        out_type=jax.ShapeDtypeStruct((2 * N, HH), jnp.float32),
        mesh=mesh,
        scratch_types=[
            pltpu.VMEM((SB, CH), jnp.int32),
            pltpu.VMEM((SB, CH), jnp.int32),
            pltpu.VMEM((CH, HH), jnp.float32),
            pltpu.VMEM((CH, HH), jnp.float32),
            pltpu.VMEM((CH, HH), jnp.float32),
            pltpu.VMEM((CH, HH), jnp.float32),
            pltpu.VMEM_SHARED((ACC_R, HH), jnp.float32),
        ] + [pltpu.SemaphoreType.DMA] * 8,
        compiler_params=pltpu.CompilerParams(use_tc_tiling_on_sc=False),
    )(x2, srcI, dstI, zs)


def _dense_body(beta, alo, ahi, xlo, xhi, w_ref, o_ref):
    c = pl.program_id(1)
    s_lo = (1.0 - ALPHA) * alo[...] + ALPHA * xlo[...]
    s_hi = (1.0 - ALPHA) * ahi[...] + ALPHA * xhi[...]
    s = jnp.concatenate([s_lo, s_hi], axis=1)
    w = w_ref[...]
    wh = jnp.where(c == 0, w[:, :HH], w[:, HH:])
    sh = jnp.where(c == 0, s_lo, s_hi)
    t = (1.0 - beta) * sh + beta * _dot(s, wh)
    o_ref[...] = jnp.maximum(t, 0.0)


def _dense(agg2, x02, w, beta):
    body = functools.partial(_dense_body, beta)
    return pl.pallas_call(
        body,
        grid=(NB, NSC),
        in_specs=[
            pl.BlockSpec((R, HH), lambda rb, c: (rb, 0)),
            pl.BlockSpec((R, HH), lambda rb, c: (NB + rb, 0)),
            pl.BlockSpec((R, HH), lambda rb, c: (rb, 0)),
            pl.BlockSpec((R, HH), lambda rb, c: (NB + rb, 0)),
            pl.BlockSpec((H, H), lambda rb, c: (0, 0)),
        ],
        out_specs=pl.BlockSpec((R, HH), lambda rb, c: (c * NB + rb, 0)),
        out_shape=jax.ShapeDtypeStruct((2 * N, HH), jnp.float32),
    )(agg2, agg2, x02, x02, w)


def _final_body(xlo, xhi, w_ref, b_ref, batch_ref, gr_ref, m1w, m1b, g1, c1,
                m2w, m2b, g2, c2, m3w, m3b, o_ref, pooled):
    rb = pl.program_id(0)
    x = jnp.concatenate([xlo[...], xhi[...]], axis=1)
    y = _dot(x, w_ref[...]) + b_ref[...]
    bid = batch_ref[0]
    gi = lax.broadcasted_iota(jnp.int32, (1, G), 1)
    mask = bid == gi
    neg = jnp.float32(-jnp.inf)

    @pl.when(rb == 0)
    def _():
        pooled[...] = jnp.full((G, H), neg, jnp.float32)

    gmin = gr_ref[0, 0, 0]
    gmax = gr_ref[0, 0, 1]
    for g in range(G):
        @pl.when((gmin <= g) & (g <= gmax))
        def _():
            mg = mask[:, g:g + 1]
            bm = jnp.max(jnp.where(mg, y, neg), axis=0, keepdims=True)
            pooled[g:g + 1, :] = jnp.maximum(pooled[g:g + 1, :], bm)

    @pl.when(rb == 0)
    def _():
        o_ref[...] = jnp.zeros_like(o_ref)

    @pl.when(rb == NB - 1)
    def _():
        p = pooled[...]

        def bn(v, gg, bb):
            m = jnp.mean(v, axis=0, keepdims=True)
            var = jnp.mean((v - m) ** 2, axis=0, keepdims=True)
            return (v - m) / jnp.sqrt(var + 1e-5) * gg[...] + bb[...]

        h1 = jnp.maximum(bn(_dot(p, m1w[...])
                            + m1b[...], g1, c1), 0.0)
        h2 = jnp.maximum(bn(_dot(h1, m2w[...])
                            + m2b[...], g2, c2), 0.0)
        o = _dot(h2, m3w[...]) + m3b[...]
        mx = jnp.max(o, axis=1, keepdims=True)
        lse = jnp.log(jnp.sum(jnp.exp(o - mx), axis=1, keepdims=True)) + mx
        o_ref[...] = o - lse


def _final(x2, batch3, granges, lin1_W, lin1_b, m1w, m1b, g1, c1,
           m2w, m2b, g2, c2, m3w, m3b):
    full = lambda a, b: pl.BlockSpec((a, b), lambda rb: (0, 0))
    return pl.pallas_call(
        _final_body,
        grid=(NB,),
        in_specs=[
            pl.BlockSpec((R, HH), lambda rb: (rb, 0)),
            pl.BlockSpec((R, HH), lambda rb: (NB + rb, 0)),
            full(H, H),
            full(1, H),
            pl.BlockSpec((1, R, 1), lambda rb: (rb, 0, 0)),
            pl.BlockSpec((1, 1, 2), lambda rb: (rb, 0, 0)),
            full(H, H), full(1, H), full(1, H), full(1, H),
            full(H, H), full(1, H), full(1, H), full(1, H),
            full(H, 10), full(1, 10),
        ],
        out_specs=pl.BlockSpec((G, 10), lambda rb: (0, 0)),
        out_shape=jax.ShapeDtypeStruct((G, 10), jnp.float32),
        scratch_shapes=[pltpu.VMEM((G, H), jnp.float32)],
    )(x2, x2, lin1_W, lin1_b.reshape(1, H), batch3, granges,
      m1w, m1b.reshape(1, H), g1.reshape(1, H), c1.reshape(1, H),
      m2w, m2b.reshape(1, H), g2.reshape(1, H), c2.reshape(1, H),
      m3w, m3b.reshape(1, 10))


def kernel(pos, edge_index, batch, lin0_W, lin0_b, conv_W, lin1_W, lin1_b,
           mlp1_W, mlp1_b, bn1_g, bn1_b, mlp2_W, mlp2_b, bn2_g, bn2_b,
           mlp3_W, mlp3_b):
    src = edge_index[0]
    dst = edge_index[1]
    pad = E_PAD - E
    srcp = jnp.concatenate([src, jnp.zeros((pad,), jnp.int32)])
    dstp = jnp.concatenate([dst, jnp.full((pad,), N, jnp.int32)])
    dstp, srcp = lax.sort((dstp, srcp), num_keys=1, is_stable=True)
    srcI = jnp.stack([srcp, srcp + N]).reshape(NSC, NSUB, CT, CH)
    dstI = dstp.reshape(NSUB, CT, CH)
    zs = jnp.zeros((ZR, HH), jnp.float32)
    batch3 = batch.reshape(NB, R, 1)
    granges = jnp.stack([batch[::R], batch[R - 1::R]], axis=1).reshape(NB, 1, 2)

    x02 = _lin0(pos, lin0_W, lin0_b)
    x2 = x02
    for layer in range(NLAYERS):
        beta = float(np.log(THETA / (layer + 1) + 1.0))
        agg2 = _segsum(x2, srcI, dstI, zs)
        x2 = _dense(agg2, x02, conv_W[layer], beta)
    return _final(x2, batch3, granges, lin1_W, lin1_b, mlp1_W, mlp1_b,
                  bn1_g, bn1_b, mlp2_W, mlp2_b, bn2_g, bn2_b, mlp3_W, mlp3_b)
